# baseline (speedup 1.0000x reference)
import functools

import jax
import jax.numpy as jnp
from jax import lax
from jax.experimental import pallas as pl
from jax.experimental.pallas import tpu as pltpu, tpu_sc as plsc

_BATCH = 16384
_DIM = 32
_LANES = 16
_NC = 2
_NS = 16
_NW = _NC * _NS
_BPW = _BATCH // _NW
_CHUNK = 128
_NCHUNK = _BPW // _CHUNK


def _sc_body(user_table, job_table, uidx_hbm, jidx_hbm, out_hbm,
             uidx_v, jidx_v, urows_v, jrows_v, out_v, sem):
    wid = lax.axis_index("s") * _NC + lax.axis_index("c")
    row0 = wid * _NCHUNK
    base = wid * _BPW

    pltpu.sync_copy(uidx_hbm.at[pl.ds(row0, _NCHUNK)], uidx_v)
    pltpu.sync_copy(jidx_hbm.at[pl.ds(row0, _NCHUNK)], jidx_v)

    copies = []
    for k in range(_NCHUNK):
        dst = pl.ds(k * _CHUNK, _CHUNK)
        copies.append(pltpu.async_copy(
            user_table.at[uidx_v.at[k]], urows_v.at[dst], sem))
        copies.append(pltpu.async_copy(
            job_table.at[jidx_v.at[k]], jrows_v.at[dst], sem))
    for c in copies:
        c.wait()

    p = urows_v[...] * jrows_v[...]
    w = _DIM
    while w > 1:
        w //= 2
        p = p[:, :w] + p[:, w:]
    out_v[...] = p

    pltpu.sync_copy(out_v, out_hbm.at[pl.ds(base, _BPW)])


@functools.partial(
    pl.kernel,
    out_type=jax.ShapeDtypeStruct((_BATCH, 1), jnp.float32),
    mesh=plsc.VectorSubcoreMesh(core_axis_name="c", subcore_axis_name="s"),
    compiler_params=pltpu.CompilerParams(use_tc_tiling_on_sc=False),
    scratch_types=[
        pltpu.VMEM((_NCHUNK, _CHUNK), jnp.int32),
        pltpu.VMEM((_NCHUNK, _CHUNK), jnp.int32),
        pltpu.VMEM((_BPW, _DIM), jnp.float32),
        pltpu.VMEM((_BPW, _DIM), jnp.float32),
        pltpu.VMEM((_BPW, 1), jnp.float32),
        pltpu.SemaphoreType.DMA,
    ],
)
def _sc_kernel(user_table, job_table, uidx, jidx, out,
               uidx_v, jidx_v, urows_v, jrows_v, out_v, sem):
    _sc_body(user_table, job_table, uidx, jidx, out,
             uidx_v, jidx_v, urows_v, jrows_v, out_v, sem)


def kernel(user, job, user_table, job_table):
    user = user.astype(jnp.int32).reshape(_BATCH // _CHUNK, _CHUNK)
    job = job.astype(jnp.int32).reshape(_BATCH // _CHUNK, _CHUNK)
    return _sc_kernel(user_table, job_table, user, job)
